# Initial kernel scaffold; baseline (speedup 1.0000x reference)
import jax
import jax.numpy as jnp
from jax.experimental import pallas as pl


def kernel(feature, edge_index, edge_weight, W1, b1, W2, b2):
    raise NotImplementedError("write your pallas kernel here")


# speedup vs baseline: 1.1401x; 1.1401x over previous
import functools

import jax
import jax.numpy as jnp
from jax import lax
from jax.experimental import pallas as pl
from jax.experimental.pallas import tpu as pltpu
from jax.experimental.pallas import tpu_sc as plsc

N_NODES = 10000
N_EDGES = 160000
H = 32

NC = 2
NS = 16
NW = NC * NS
EPW = N_EDGES // NW
CH = 40
NCH = EPW // CH
ACC_ROWS = 10240
RPT = ACC_ROWS // NS


def _sc_kernel(feature_hbm, src_hbm, dst_hbm, w_hbm, partial_hbm,
               src_small, dst_small,
               wbuf, rows, msg, zbuf, acc, gsem):
    cid = lax.axis_index("c")
    sid = lax.axis_index("s")
    wid = sid * NC + cid

    if True:
        def zrow(r, carry):
            zbuf[r, 0:16] = jnp.zeros((16,), jnp.float32)
            zbuf[r, 16:32] = jnp.zeros((16,), jnp.float32)
            return carry
        lax.fori_loop(0, RPT, zrow, 0)
        pltpu.sync_copy(zbuf, acc.at[pl.ds(sid * RPT, RPT)])
        plsc.subcore_barrier()

        def chunk(c, carry):
            base = wid * EPW + c * CH
            pltpu.sync_copy(w_hbm.at[pl.ds(base, CH)], wbuf)
            pltpu.sync_copy(src_hbm.at[pl.ds(base, CH)], src_small)
            pltpu.sync_copy(dst_hbm.at[pl.ds(base, CH)], dst_small)
            pltpu.async_copy(feature_hbm.at[src_small], rows, gsem).wait()

            def edge(e, carry2):
                a0 = jnp.zeros((16,), jnp.float32)
                a1 = jnp.zeros((16,), jnp.float32)
                f0 = rows[e, 0:16]
                f1 = rows[e, 16:32]
                for i in range(H):
                    fi = f0[i] if i < 16 else f1[i - 16]
                    fv = jnp.full((16,), fi, jnp.float32)
                    a0 = a0 + fv * wbuf[e, i, 0:16]
                    a1 = a1 + fv * wbuf[e, i, 16:32]
                msg[e, 0:16] = a0
                msg[e, 16:32] = a1
                return carry2
            lax.fori_loop(0, CH, edge, 0)

            pltpu.sync_copy(msg, acc.at[dst_small], add=True)
            return carry
        lax.fori_loop(0, NCH, chunk, 0)

        plsc.subcore_barrier()
        pltpu.sync_copy(acc.at[pl.ds(sid * RPT, RPT)],
                        partial_hbm.at[cid, pl.ds(sid * RPT, RPT)])


def _combine_body(f_ref, p_ref, o_ref):
    o_ref[...] = 0.5 * (f_ref[...] + p_ref[0, :N_NODES] + p_ref[1, :N_NODES])


def kernel(feature, edge_index, edge_weight, W1, b1, W2, b2):
    sc = functools.partial(
        pl.kernel,
        out_type=jax.ShapeDtypeStruct((NC, ACC_ROWS, H), jnp.float32),
        mesh=plsc.VectorSubcoreMesh(core_axis_name="c", subcore_axis_name="s"),
        compiler_params=pltpu.CompilerParams(use_tc_tiling_on_sc=False),
        scratch_types=[
            pltpu.VMEM((CH,), jnp.int32),
            pltpu.VMEM((CH,), jnp.int32),
            pltpu.VMEM((CH, H, H), jnp.float32),
            pltpu.VMEM((CH, H), jnp.float32),
            pltpu.VMEM((CH, H), jnp.float32),
            pltpu.VMEM((RPT, H), jnp.float32),
            pltpu.VMEM_SHARED((ACC_ROWS, H), jnp.float32),
            pltpu.SemaphoreType.DMA,
        ],
    )(_sc_kernel)
    partial = sc(feature, edge_index[0], edge_index[1], edge_weight)

    out = pl.pallas_call(
        _combine_body,
        out_shape=jax.ShapeDtypeStruct((N_NODES, H), jnp.float32),
    )(feature, partial)
    return out


# speedup vs baseline: 1.3284x; 1.1652x over previous
import functools

import jax
import jax.numpy as jnp
from jax import lax
from jax.experimental import pallas as pl
from jax.experimental.pallas import tpu as pltpu
from jax.experimental.pallas import tpu_sc as plsc

N_NODES = 10000
N_EDGES = 160000
H = 32

NC = 2
NS = 16
NW = NC * NS
EPW = N_EDGES // NW
CH = 40
NCH = EPW // CH
ACC_ROWS = 10240
RPT = ACC_ROWS // NS
ZR = 128


def _sc_kernel(feature_hbm, src_hbm, dst_hbm, w_hbm, partial_hbm,
               src_flat, dst_buf, wbuf0, wbuf1, rows0, rows1, msg, zbuf,
               acc, wsem0, wsem1, gsem0, gsem1):
    cid = lax.axis_index("c")
    sid = lax.axis_index("s")
    wid = sid * NC + cid

    def zrow(r, carry):
        zbuf[r, 0:16] = jnp.zeros((16,), jnp.float32)
        zbuf[r, 16:32] = jnp.zeros((16,), jnp.float32)
        return carry
    lax.fori_loop(0, ZR, zrow, 0)
    for z in range(RPT // ZR):
        pltpu.sync_copy(zbuf, acc.at[pl.ds(sid * RPT + z * ZR, ZR)])
    plsc.subcore_barrier()

    pltpu.sync_copy(src_hbm.at[pl.ds(wid * EPW, EPW)], src_flat)
    pltpu.sync_copy(dst_hbm.at[wid], dst_buf)

    bufs = ((wbuf0, rows0, wsem0, gsem0), (wbuf1, rows1, wsem1, gsem1))

    def start(c, slot):
        wbuf, rows, wsem, gsem = bufs[slot]
        pltpu.async_copy(w_hbm.at[pl.ds(wid * EPW + c * CH, CH)], wbuf, wsem)
        pltpu.async_copy(
            feature_hbm.at[src_flat.at[pl.ds(c * CH, CH)]], rows, gsem)

    def do_chunk(c, slot):
        wbuf, rows, wsem, gsem = bufs[slot]
        pltpu.make_async_copy(
            w_hbm.at[pl.ds(wid * EPW + c * CH, CH)], wbuf, wsem).wait()
        pltpu.make_async_copy(
            feature_hbm.at[src_flat.at[pl.ds(c * CH, CH)]], rows, gsem).wait()

        def edge(e, carry2):
            a0 = jnp.zeros((16,), jnp.float32)
            a1 = jnp.zeros((16,), jnp.float32)
            f0 = rows[e, 0:16]
            f1 = rows[e, 16:32]
            for i in range(H):
                fi = f0[i] if i < 16 else f1[i - 16]
                fv = jnp.full((16,), fi, jnp.float32)
                a0 = a0 + fv * wbuf[e, i, 0:16]
                a1 = a1 + fv * wbuf[e, i, 16:32]
            msg[e, 0:16] = a0
            msg[e, 16:32] = a1
            return carry2
        lax.fori_loop(0, CH, edge, 0)

        @pl.when(c + 2 < NCH)
        def _():
            start(c + 2, slot)

        pltpu.sync_copy(msg, acc.at[dst_buf.at[c]], add=True)

    start(0, 0)
    start(1, 1)

    def pair(k, carry):
        do_chunk(2 * k, 0)
        do_chunk(2 * k + 1, 1)
        return carry
    lax.fori_loop(0, NCH // 2, pair, 0)
    do_chunk(NCH - 1, 0)

    plsc.subcore_barrier()
    pltpu.sync_copy(acc.at[pl.ds(sid * RPT, RPT)],
                    partial_hbm.at[cid, pl.ds(sid * RPT, RPT)])


def _combine_body(f_ref, p_ref, o_ref):
    o_ref[...] = 0.5 * (f_ref[...] + p_ref[0, :N_NODES] + p_ref[1, :N_NODES])


def kernel(feature, edge_index, edge_weight, W1, b1, W2, b2):
    sc = functools.partial(
        pl.kernel,
        out_type=jax.ShapeDtypeStruct((NC, ACC_ROWS, H), jnp.float32),
        mesh=plsc.VectorSubcoreMesh(core_axis_name="c", subcore_axis_name="s"),
        compiler_params=pltpu.CompilerParams(use_tc_tiling_on_sc=False),
        scratch_types=[
            pltpu.VMEM((EPW,), jnp.int32),
            pltpu.VMEM((NCH, CH), jnp.int32),
            pltpu.VMEM((CH, H, H), jnp.float32),
            pltpu.VMEM((CH, H, H), jnp.float32),
            pltpu.VMEM((CH, H), jnp.float32),
            pltpu.VMEM((CH, H), jnp.float32),
            pltpu.VMEM((CH, H), jnp.float32),
            pltpu.VMEM((ZR, H), jnp.float32),
            pltpu.VMEM_SHARED((ACC_ROWS, H), jnp.float32),
            pltpu.SemaphoreType.DMA,
            pltpu.SemaphoreType.DMA,
            pltpu.SemaphoreType.DMA,
            pltpu.SemaphoreType.DMA,
        ],
    )(_sc_kernel)
    partial = sc(feature, edge_index[0],
                 edge_index[1].reshape(NW, NCH, CH), edge_weight)

    out = pl.pallas_call(
        _combine_body,
        out_shape=jax.ShapeDtypeStruct((N_NODES, H), jnp.float32),
    )(feature, partial)
    return out


# speedup vs baseline: 4.5899x; 3.4551x over previous
import functools

import jax
import jax.numpy as jnp
from jax import lax
from jax.experimental import pallas as pl
from jax.experimental.pallas import tpu as pltpu
from jax.experimental.pallas import tpu_sc as plsc

N_NODES = 10000
N_EDGES = 160000
H = 32

NC = 2
NS = 16
NW = NC * NS
CHE = 128
NCHG = N_EDGES // CHE
EHI = NCHG
ACC_ROWS = 10240
RPT = ACC_ROWS // NS
NPASS = 4


def _i16(v):
    return jnp.full((16,), v, jnp.int32)


def _sc_kernel(feature_hbm, src_hbm, dst_hbm, w_hbm, partial_hbm,
               src0, src1, dst0, dst1, rows0, rows1, wbuf0, wbuf1,
               msgT, msg, acc,
               ssem0, ssem1, dsem0, dsem1, gsem0, gsem1, wsem0, wsem1):
    cid = lax.axis_index("c")
    sid = lax.axis_index("s")
    wid = sid * NC + cid
    nch = 39 + jnp.where(wid < 2, 1, 0)

    iota = jnp.arange(16, dtype=jnp.int32)

    def zrow(r, carry):
        msg[r, 0:16] = jnp.zeros((16,), jnp.float32)
        msg[r, 16:32] = jnp.zeros((16,), jnp.float32)
        return carry
    lax.fori_loop(0, CHE, zrow, 0)
    for z in range(RPT // CHE):
        pltpu.sync_copy(msg, acc.at[pl.ds(sid * RPT + z * CHE, CHE)])
    plsc.subcore_barrier()

    srcs = (src0, src1)
    dsts = (dst0, dst1)
    rows = (rows0, rows1)
    wbufs = (wbuf0, wbuf1)
    ssems = (ssem0, ssem1)
    dsems = (dsem0, dsem1)
    gsems = (gsem0, gsem1)
    wsems = (wsem0, wsem1)

    def chunk_of(t):
        return wid + NW * t

    def start_idx(t, slot):
        b = chunk_of(t)
        pltpu.async_copy(src_hbm.at[pl.ds(b * CHE, CHE)], srcs[slot],
                         ssems[slot])
        pltpu.async_copy(dst_hbm.at[pl.ds(b * CHE, CHE)], dsts[slot],
                         dsems[slot])

    def start_gather(t, slot):
        pltpu.make_async_copy(
            src_hbm.at[pl.ds(chunk_of(t) * CHE, CHE)], srcs[slot],
            ssems[slot]).wait()
        pltpu.async_copy(feature_hbm.at[srcs[slot]], rows[slot], gsems[slot])

    def start_w(t, p, wslot):
        pltpu.async_copy(w_hbm.at[:, p, chunk_of(t)], wbufs[wslot],
                         wsems[wslot])

    def wait_w(t, p, wslot):
        pltpu.make_async_copy(
            w_hbm.at[:, p, chunk_of(t)], wbufs[wslot], wsems[wslot]).wait()

    def do_chunk(t, slot):
        rbuf = rows[slot]
        pltpu.make_async_copy(
            feature_hbm.at[srcs[slot]], rbuf, gsems[slot]).wait()

        for p in range(NPASS):
            wslot = p & 1
            wait_w(t, p, wslot)
            wbuf = wbufs[wslot]

            def group(g, carry):
                e_vec = iota + 16 * g
                accs = [jnp.zeros((16,), jnp.float32) for _ in range(8)]
                for i in range(H):
                    f_i = plsc.load_gather(rbuf, [e_vec, _i16(i)])
                    for jl in range(8):
                        accs[jl] = accs[jl] + f_i * wbuf[i, jl,
                                                         pl.ds(16 * g, 16)]
                for jl in range(8):
                    msgT[8 * p + jl, pl.ds(16 * g, 16)] = accs[jl]
                return carry
            lax.fori_loop(0, CHE // 16, group, 0)

            if p < 2:
                start_w(t, p + 2, wslot)
            else:
                @pl.when(t + 1 < nch)
                def _():
                    start_w(t + 1, p - 2, wslot)

            if p == 0:
                @pl.when(t + 1 < nch)
                def _():
                    start_idx(t + 1, slot ^ 1)
            if p == 1:
                @pl.when(t + 1 < nch)
                def _():
                    start_gather(t + 1, slot ^ 1)

        def tr(e, carry):
            ev = _i16(e)
            r0 = plsc.load_gather(msgT, [iota, ev])
            r1 = plsc.load_gather(msgT, [iota + 16, ev])
            msg[e, 0:16] = r0
            msg[e, 16:32] = r1
            return carry
        lax.fori_loop(0, CHE, tr, 0)

        pltpu.make_async_copy(
            dst_hbm.at[pl.ds(chunk_of(t) * CHE, CHE)], dsts[slot],
            dsems[slot]).wait()
        pltpu.sync_copy(msg, acc.at[dsts[slot]], add=True)

    start_idx(0, 0)
    start_gather(0, 0)
    start_w(0, 0, 0)
    start_w(0, 1, 1)

    def pairs(tt, carry):
        @pl.when(2 * tt < nch)
        def _():
            do_chunk(2 * tt, 0)

        @pl.when(2 * tt + 1 < nch)
        def _():
            do_chunk(2 * tt + 1, 1)
        return carry
    lax.fori_loop(0, 20, pairs, 0)

    plsc.subcore_barrier()
    pltpu.sync_copy(acc.at[pl.ds(sid * RPT, RPT)],
                    partial_hbm.at[cid, pl.ds(sid * RPT, RPT)])


def _combine_body(f_ref, p_ref, o_ref):
    o_ref[...] = 0.5 * (f_ref[...] + p_ref[0, :N_NODES] + p_ref[1, :N_NODES])


def kernel(feature, edge_index, edge_weight, W1, b1, W2, b2):
    sc = functools.partial(
        pl.kernel,
        out_type=jax.ShapeDtypeStruct((NC, ACC_ROWS, H), jnp.float32),
        mesh=plsc.VectorSubcoreMesh(core_axis_name="c", subcore_axis_name="s"),
        compiler_params=pltpu.CompilerParams(use_tc_tiling_on_sc=False,
                                             needs_layout_passes=False),
        scratch_types=[
            pltpu.VMEM((CHE,), jnp.int32),
            pltpu.VMEM((CHE,), jnp.int32),
            pltpu.VMEM((CHE,), jnp.int32),
            pltpu.VMEM((CHE,), jnp.int32),
            pltpu.VMEM((CHE, H), jnp.float32),
            pltpu.VMEM((CHE, H), jnp.float32),
            pltpu.VMEM((H, 8, CHE), jnp.float32),
            pltpu.VMEM((H, 8, CHE), jnp.float32),
            pltpu.VMEM((H, CHE), jnp.float32),
            pltpu.VMEM((CHE, H), jnp.float32),
            pltpu.VMEM_SHARED((ACC_ROWS, H), jnp.float32),
            pltpu.SemaphoreType.DMA,
            pltpu.SemaphoreType.DMA,
            pltpu.SemaphoreType.DMA,
            pltpu.SemaphoreType.DMA,
            pltpu.SemaphoreType.DMA,
            pltpu.SemaphoreType.DMA,
            pltpu.SemaphoreType.DMA,
            pltpu.SemaphoreType.DMA,
        ],
    )(_sc_kernel)
    w5 = (edge_weight.transpose(1, 2, 0)
          .reshape(H, 4, 8, EHI, CHE)
          .transpose(0, 1, 3, 2, 4))
    partial = sc(feature, edge_index[0], edge_index[1], w5)

    out = pl.pallas_call(
        _combine_body,
        out_shape=jax.ShapeDtypeStruct((N_NODES, H), jnp.float32),
    )(feature, partial)
    return out
